# baseline (speedup 1.0000x reference)
import functools

import jax
import jax.numpy as jnp
from jax import lax
from jax.experimental import pallas as pl
from jax.experimental.pallas import tpu as pltpu
from jax.experimental.pallas import tpu_sc as plsc

VOCAB = 1000000
D_MODEL = 64
BATCH = 16384
HIST = 20
NUM_WORKERS = 32
TBLK = 7936
SPLIT = 499968
H_ROWS = 500032
NBT = BATCH // 128
UNITS = HIST * NBT
UNITS_PER_W = UNITS // NUM_WORKERS
IDX_PER_W = UNITS_PER_W * 128


def _transpose_body(t1_ref, t2_ref, o_ref):
    o_ref[...] = jnp.concatenate([t1_ref[...].T, t2_ref[...].T], axis=1)


_transpose = pl.pallas_call(
    _transpose_body,
    grid=(pl.cdiv(H_ROWS, TBLK),),
    in_specs=[
        pl.BlockSpec((D_MODEL, TBLK), lambda i: (0, i)),
        pl.BlockSpec((D_MODEL, TBLK), lambda i: (0, i + SPLIT // TBLK)),
    ],
    out_specs=pl.BlockSpec((TBLK, 128), lambda i: (i, 0)),
    out_shape=jax.ShapeDtypeStruct((H_ROWS, 128), jnp.float32),
)


def _make_gather():
    mesh = plsc.VectorSubcoreMesh(core_axis_name="c", subcore_axis_name="s")

    @functools.partial(
        pl.kernel,
        mesh=mesh,
        out_type=jax.ShapeDtypeStruct((HIST, 8, NBT, 8, 128), jnp.float32),
        compiler_params=pltpu.CompilerParams(
            use_tc_tiling_on_sc=False, needs_layout_passes=False
        ),
        scratch_types=[
            pltpu.VMEM((IDX_PER_W,), jnp.int32),
            pltpu.VMEM((IDX_PER_W,), jnp.int32),
            pltpu.VMEM((2, 128, D_MODEL), jnp.float32),
            pltpu.VMEM((2, 8, 8, 128), jnp.float32),
            pltpu.SemaphoreType.DMA,
            pltpu.SemaphoreType.DMA,
            pltpu.SemaphoreType.DMA,
            pltpu.SemaphoreType.DMA,
            pltpu.SemaphoreType.DMA,
        ],
    )
    def gather(table_hbm, idx_hbm, out_hbm, idx_v, idx2_v, rows_v, tile_v,
               sem_i, sem_g0, sem_g1, sem_o0, sem_o1):
        wid = lax.axis_index("s") * 2 + lax.axis_index("c")
        g0 = wid * UNITS_PER_W
        sem_g = (sem_g0, sem_g1)
        sem_o = (sem_o0, sem_o1)

        pltpu.async_copy(
            idx_hbm.at[pl.ds(g0 * 128, IDX_PER_W)], idx_v, sem_i
        ).wait()

        def remap_step(g, carry):
            v = idx_v[pl.ds(g * 16, 16)]
            u = v + v
            idx2_v[pl.ds(g * 16, 16)] = jnp.where(
                v < SPLIT, u, u - (2 * SPLIT - 1)
            )
            return carry

        lax.fori_loop(0, IDX_PER_W // 16, remap_step, 0)

        def fire_gather(u, p):
            pltpu.async_copy(
                table_hbm.at[idx2_v.at[pl.ds(u * 128, 128)]],
                rows_v.at[p], sem_g[p],
            )

        def wait_gather(u, p):
            pltpu.make_async_copy(
                table_hbm.at[idx2_v.at[pl.ds(u * 128, 128)]],
                rows_v.at[p], sem_g[p],
            ).wait()

        iotas = [jnp.arange(16, dtype=jnp.int32) + 16 * g for g in range(8)]

        def transpose(p):
            for R in range(8):
                for r in range(8):
                    col = jnp.full((16,), 8 * R + r, jnp.int32)
                    for g in range(8):
                        tile_v[p, R, r, pl.ds(16 * g, 16)] = plsc.load_gather(
                            rows_v.at[p], [iotas[g], col]
                        )

        def fire_write(u, p):
            g = g0 + u
            h = g // NBT
            bt = g % NBT
            for R in range(8):
                pltpu.async_copy(
                    tile_v.at[p, R], out_hbm.at[h, R, bt], sem_o[p]
                )

        def wait_write(u, p):
            g = g0 + u
            h = g // NBT
            bt = g % NBT
            for R in range(8):
                pltpu.make_async_copy(
                    tile_v.at[p, R], out_hbm.at[h, R, bt], sem_o[p]
                ).wait()

        fire_gather(0, 0)

        def pair(k, carry):
            u0 = 2 * k
            u1 = u0 + 1
            fire_gather(u1, 1)
            wait_gather(u0, 0)

            @pl.when(k >= 1)
            def _():
                wait_write(u0 - 2, 0)

            transpose(0)
            fire_write(u0, 0)

            @pl.when(k + 1 < UNITS_PER_W // 2)
            def _():
                fire_gather(u0 + 2, 0)

            wait_gather(u1, 1)

            @pl.when(k >= 1)
            def _():
                wait_write(u1 - 2, 1)

            transpose(1)
            fire_write(u1, 1)
            return carry

        lax.fori_loop(0, UNITS_PER_W // 2, pair, 0)
        wait_write(UNITS_PER_W - 2, 0)
        wait_write(UNITS_PER_W - 1, 1)

    return gather


@jax.jit
def kernel(x, embedding_matrix):
    idx = x.T.reshape(BATCH * HIST).astype(jnp.int32)
    t_view = embedding_matrix.T
    table_h = _transpose(t_view, t_view)
    table_rm = table_h.reshape(2 * H_ROWS, D_MODEL)
    out5 = _make_gather()(table_rm, idx)
    return out5.transpose(2, 4, 0, 1, 3).reshape(BATCH, HIST, D_MODEL)


# speedup vs baseline: 1.2777x; 1.2777x over previous
import functools

import jax
import jax.numpy as jnp
from jax import lax
from jax.experimental import pallas as pl
from jax.experimental.pallas import tpu as pltpu
from jax.experimental.pallas import tpu_sc as plsc

VOCAB = 1000000
D_MODEL = 64
BATCH = 16384
HIST = 20
NUM_WORKERS = 32
CHUNK = 1024
TBLK = 7936
SPLIT = 499968
H_ROWS = 500032
NBT = BATCH // 128
IDX_PER_W = BATCH * HIST // NUM_WORKERS


def _transpose_body(t1_ref, t2_ref, o_ref):
    o_ref[...] = jnp.concatenate([t1_ref[...].T, t2_ref[...].T], axis=1)


_transpose = pl.pallas_call(
    _transpose_body,
    grid=(pl.cdiv(H_ROWS, TBLK),),
    in_specs=[
        pl.BlockSpec((D_MODEL, TBLK), lambda i: (0, i)),
        pl.BlockSpec((D_MODEL, TBLK), lambda i: (0, i + SPLIT // TBLK)),
    ],
    out_specs=pl.BlockSpec((TBLK, 128), lambda i: (i, 0)),
    out_shape=jax.ShapeDtypeStruct((H_ROWS, 128), jnp.float32),
)


def _format_body(in_ref, o_ref):
    for t in range(16):
        a = in_ref[0, t * 64:(t + 1) * 64, :]
        c = jnp.concatenate([a[:, :64].T, a[:, 64:].T], axis=1)
        for R in range(8):
            o_ref[0, R, t, :, :] = c[R * 8:(R + 1) * 8, :]


_format = pl.pallas_call(
    _format_body,
    grid=(HIST, NBT // 16),
    in_specs=[pl.BlockSpec((1, 1024, 128), lambda h, j: (h, j, 0))],
    out_specs=pl.BlockSpec((1, 8, 16, 8, 128), lambda h, j: (h, 0, j, 0, 0)),
    out_shape=jax.ShapeDtypeStruct((HIST, 8, NBT, 8, 128), jnp.float32),
)


def _make_gather():
    mesh = plsc.VectorSubcoreMesh(core_axis_name="c", subcore_axis_name="s")

    @functools.partial(
        pl.kernel,
        mesh=mesh,
        out_type=jax.ShapeDtypeStruct((BATCH * HIST, D_MODEL), jnp.float32),
        compiler_params=pltpu.CompilerParams(
            use_tc_tiling_on_sc=False, needs_layout_passes=False
        ),
        scratch_types=[
            pltpu.VMEM((IDX_PER_W,), jnp.int32),
            pltpu.VMEM((IDX_PER_W,), jnp.int32),
            pltpu.VMEM((CHUNK, D_MODEL), jnp.float32),
            pltpu.SemaphoreType.DMA,
            pltpu.SemaphoreType.DMA,
        ],
    )
    def gather(table_hbm, idx_hbm, out_hbm, idx_v, idx2_v, rows_v,
               sem_i, sem_g):
        wid = lax.axis_index("s") * 2 + lax.axis_index("c")
        base = wid * IDX_PER_W

        pltpu.async_copy(
            idx_hbm.at[pl.ds(base, IDX_PER_W)], idx_v, sem_i
        ).wait()

        iota = jnp.arange(16, dtype=jnp.int32)

        def remap_step(g, carry):
            j = iota + g * 16
            v = idx_v[pl.ds(g * 16, 16)]
            u = v + v
            u = jnp.where(v < SPLIT, u, u - (2 * SPLIT - 1))
            jj = lax.rem(j, 128)
            q = jj + jj - jnp.where(jj < 64, 0, 127)
            plsc.store_scatter(idx2_v, [j - jj + q], u)
            return carry

        lax.fori_loop(0, IDX_PER_W // 16, remap_step, 0)

        def body(i, carry):
            off = base + i * CHUNK
            pltpu.async_copy(
                table_hbm.at[idx2_v.at[pl.ds(i * CHUNK, CHUNK)]],
                rows_v, sem_g,
            ).wait()
            pltpu.sync_copy(rows_v, out_hbm.at[pl.ds(off, CHUNK)])
            return carry

        lax.fori_loop(0, IDX_PER_W // CHUNK, body, 0)

    return gather


@jax.jit
def kernel(x, embedding_matrix):
    idx = x.T.reshape(BATCH * HIST).astype(jnp.int32)
    t_view = embedding_matrix.T
    table_h = _transpose(t_view, t_view)
    table_rm = table_h.reshape(2 * H_ROWS, D_MODEL)
    rows = _make_gather()(table_rm, idx)
    out5 = _format(rows.reshape(HIST, BATCH // 2, 128))
    return out5.transpose(2, 4, 0, 1, 3).reshape(BATCH, HIST, D_MODEL)


# speedup vs baseline: 2.4272x; 1.8997x over previous
import functools

import jax
import jax.numpy as jnp
from jax import lax
from jax.experimental import pallas as pl
from jax.experimental.pallas import tpu as pltpu
from jax.experimental.pallas import tpu_sc as plsc

VOCAB = 1000000
D_MODEL = 64
BATCH = 16384
HIST = 20
NUM_WORKERS = 32
CHUNK = 1024
TBLK = 7936
SPLIT = 499968
H_ROWS = 500032
NBT = BATCH // 128
IDX_PER_W = BATCH * HIST // NUM_WORKERS


def _transpose_body(t1_ref, t2_ref, o_ref):
    o_ref[...] = jnp.concatenate([t1_ref[...], t2_ref[...]], axis=0).T


_transpose = pl.pallas_call(
    _transpose_body,
    grid=(pl.cdiv(H_ROWS, TBLK),),
    in_specs=[
        pl.BlockSpec((D_MODEL, TBLK), lambda i: (0, i)),
        pl.BlockSpec((D_MODEL, TBLK), lambda i: (0, i + SPLIT // TBLK)),
    ],
    out_specs=pl.BlockSpec((TBLK, 128), lambda i: (i, 0)),
    out_shape=jax.ShapeDtypeStruct((H_ROWS, 128), jnp.float32),
)


def _format_body(in_ref, o_ref):
    for t in range(64):
        at = in_ref[0, t * 64:(t + 1) * 64, :].T
        c = jnp.concatenate([at[:64], at[64:]], axis=1)
        for R in range(8):
            o_ref[0, R, t, :, :] = c[R * 8:(R + 1) * 8, :]


_format = pl.pallas_call(
    _format_body,
    grid=(HIST, NBT // 64),
    in_specs=[pl.BlockSpec((1, 4096, 128), lambda h, j: (h, j, 0))],
    out_specs=pl.BlockSpec((1, 8, 64, 8, 128), lambda h, j: (h, 0, j, 0, 0)),
    out_shape=jax.ShapeDtypeStruct((HIST, 8, NBT, 8, 128), jnp.float32),
)


def _make_gather():
    mesh = plsc.VectorSubcoreMesh(core_axis_name="c", subcore_axis_name="s")

    @functools.partial(
        pl.kernel,
        mesh=mesh,
        out_type=jax.ShapeDtypeStruct((BATCH * HIST, D_MODEL), jnp.float32),
        compiler_params=pltpu.CompilerParams(
            use_tc_tiling_on_sc=False, needs_layout_passes=False
        ),
        scratch_types=[
            pltpu.VMEM((IDX_PER_W,), jnp.int32),
            pltpu.VMEM((IDX_PER_W,), jnp.int32),
            pltpu.VMEM((CHUNK, D_MODEL), jnp.float32),
            pltpu.SemaphoreType.DMA,
            pltpu.SemaphoreType.DMA,
        ],
    )
    def gather(table_hbm, idx_hbm, out_hbm, idx_v, idx2_v, rows_v,
               sem_i, sem_g):
        wid = lax.axis_index("s") * 2 + lax.axis_index("c")
        base = wid * IDX_PER_W

        pltpu.async_copy(
            idx_hbm.at[pl.ds(base, IDX_PER_W)], idx_v, sem_i
        ).wait()

        iota = jnp.arange(16, dtype=jnp.int32)

        def remap_step(g, carry):
            j = iota + g * 16
            v = idx_v[pl.ds(g * 16, 16)]
            u = v + v
            u = jnp.where(v < SPLIT, u, u - (2 * SPLIT - 1))
            jj = lax.rem(j, 128)
            q = jj + jj - jnp.where(jj < 64, 0, 127)
            plsc.store_scatter(idx2_v, [j - jj + q], u)
            return carry

        lax.fori_loop(0, IDX_PER_W // 16, remap_step, 0)

        def body(i, carry):
            off = base + i * CHUNK
            pltpu.async_copy(
                table_hbm.at[idx2_v.at[pl.ds(i * CHUNK, CHUNK)]],
                rows_v, sem_g,
            ).wait()
            pltpu.sync_copy(rows_v, out_hbm.at[pl.ds(off, CHUNK)])
            return carry

        lax.fori_loop(0, IDX_PER_W // CHUNK, body, 0)

    return gather


@jax.jit
def kernel(x, embedding_matrix):
    idx = x.T.reshape(BATCH * HIST).astype(jnp.int32)
    t_view = embedding_matrix.T
    table_h = _transpose(t_view, t_view)
    table_rm = table_h.reshape(2 * H_ROWS, D_MODEL)
    rows = _make_gather()(table_rm, idx)
    out5 = _format(rows.reshape(HIST, BATCH // 2, 128))
    return out5.transpose(2, 4, 0, 1, 3).reshape(BATCH, HIST, D_MODEL)


# speedup vs baseline: 2.4896x; 1.0257x over previous
import functools

import jax
import jax.numpy as jnp
from jax import lax
from jax.experimental import pallas as pl
from jax.experimental.pallas import tpu as pltpu
from jax.experimental.pallas import tpu_sc as plsc

VOCAB = 1000000
D_MODEL = 64
BATCH = 16384
HIST = 20
NUM_WORKERS = 32
CHUNK = 256
NBUF = 4
TBLK = 16128
SPLIT = 499968
H_ROWS = 500032
NBT = BATCH // 128
IDX_PER_W = BATCH * HIST // NUM_WORKERS


def _transpose_body(t1_ref, t2_ref, o_ref):
    o_ref[...] = jnp.concatenate([t1_ref[...], t2_ref[...]], axis=0).T


_transpose = pl.pallas_call(
    _transpose_body,
    grid=(pl.cdiv(H_ROWS, TBLK),),
    in_specs=[
        pl.BlockSpec((D_MODEL, TBLK), lambda i: (0, i)),
        pl.BlockSpec((D_MODEL, TBLK), lambda i: (0, i + SPLIT // TBLK)),
    ],
    out_specs=pl.BlockSpec((TBLK, 128), lambda i: (i, 0)),
    out_shape=jax.ShapeDtypeStruct((H_ROWS, 128), jnp.float32),
)


def _format_body(in_ref, o_ref):
    for t in range(64):
        at = in_ref[0, t * 64:(t + 1) * 64, :].T
        c = jnp.concatenate([at[:64], at[64:]], axis=1)
        for R in range(8):
            o_ref[0, R, t, :, :] = c[R * 8:(R + 1) * 8, :]


_format = pl.pallas_call(
    _format_body,
    grid=(HIST, NBT // 64),
    in_specs=[pl.BlockSpec((1, 4096, 128), lambda h, j: (h, j, 0))],
    out_specs=pl.BlockSpec((1, 8, 64, 8, 128), lambda h, j: (h, 0, j, 0, 0)),
    out_shape=jax.ShapeDtypeStruct((HIST, 8, NBT, 8, 128), jnp.float32),
)


def _make_gather():
    mesh = plsc.VectorSubcoreMesh(core_axis_name="c", subcore_axis_name="s")

    @functools.partial(
        pl.kernel,
        mesh=mesh,
        out_type=jax.ShapeDtypeStruct((BATCH * HIST, D_MODEL), jnp.float32),
        compiler_params=pltpu.CompilerParams(
            use_tc_tiling_on_sc=False, needs_layout_passes=False
        ),
        scratch_types=[
            pltpu.VMEM((IDX_PER_W,), jnp.int32),
            pltpu.VMEM((IDX_PER_W,), jnp.int32),
            pltpu.VMEM((NBUF, CHUNK, D_MODEL), jnp.float32),
            pltpu.SemaphoreType.DMA,
        ] + [pltpu.SemaphoreType.DMA] * (2 * NBUF),
    )
    def gather(table_hbm, idx_hbm, out_hbm, idx_v, idx2_v, rows_v,
               sem_i, *sems):
        sem_g = sems[:NBUF]
        sem_w = sems[NBUF:]
        wid = lax.axis_index("s") * 2 + lax.axis_index("c")
        base = wid * IDX_PER_W

        pltpu.async_copy(
            idx_hbm.at[pl.ds(base, IDX_PER_W)], idx_v, sem_i
        ).wait()

        iota = jnp.arange(16, dtype=jnp.int32)

        def remap_step(g, carry):
            j = iota + g * 16
            v = idx_v[pl.ds(g * 16, 16)]
            u = v + v
            u = jnp.where(v < SPLIT, u, u - (2 * SPLIT - 1))
            jj = lax.rem(j, 128)
            q = jj + jj - jnp.where(jj < 64, 0, 127)
            plsc.store_scatter(idx2_v, [j - jj + q], u)
            return carry

        lax.fori_loop(0, IDX_PER_W // 16, remap_step, 0)

        n_chunks = IDX_PER_W // CHUNK

        def g_copy(u, p):
            return pltpu.make_async_copy(
                table_hbm.at[idx2_v.at[pl.ds(u * CHUNK, CHUNK)]],
                rows_v.at[p], sem_g[p],
            )

        def w_copy(u, p):
            return pltpu.make_async_copy(
                rows_v.at[p], out_hbm.at[pl.ds(base + u * CHUNK, CHUNK)],
                sem_w[p],
            )

        def fire_gather(u, p):
            pltpu.async_copy(
                table_hbm.at[idx2_v.at[pl.ds(u * CHUNK, CHUNK)]],
                rows_v.at[p], sem_g[p],
            )

        def fire_write(u, p):
            pltpu.async_copy(
                rows_v.at[p], out_hbm.at[pl.ds(base + u * CHUNK, CHUNK)],
                sem_w[p],
            )

        fire_gather(0, 0)
        fire_gather(1, 1)

        def ring(k, carry):
            for j in range(NBUF):
                u = NBUF * k + j
                p = j
                pn = (j + 2) % NBUF

                @pl.when(u >= 2)
                def _():
                    w_copy(u - 2, pn).wait()

                @pl.when(u + 2 < n_chunks)
                def _():
                    fire_gather(u + 2, pn)

                g_copy(u, p).wait()
                fire_write(u, p)
            return carry

        lax.fori_loop(0, n_chunks // NBUF, ring, 0)
        w_copy(n_chunks - 2, (n_chunks - 2) % NBUF).wait()
        w_copy(n_chunks - 1, (n_chunks - 1) % NBUF).wait()

    return gather


@jax.jit
def kernel(x, embedding_matrix):
    idx = x.T.reshape(BATCH * HIST).astype(jnp.int32)
    t_view = embedding_matrix.T
    table_h = _transpose(t_view, t_view)
    table_rm = table_h.reshape(2 * H_ROWS, D_MODEL)
    rows = _make_gather()(table_rm, idx)
    out5 = _format(rows.reshape(HIST, BATCH // 2, 128))
    return out5.transpose(2, 4, 0, 1, 3).reshape(BATCH, HIST, D_MODEL)


# speedup vs baseline: 2.5732x; 1.0336x over previous
import functools

import jax
import jax.numpy as jnp
from jax import lax
from jax.experimental import pallas as pl
from jax.experimental.pallas import tpu as pltpu
from jax.experimental.pallas import tpu_sc as plsc

VOCAB = 1000000
D_MODEL = 64
BATCH = 16384
HIST = 20
NUM_WORKERS = 32
CHUNK = 256
NBUF = 4
TBLK = 16128
SPLIT = 499968
H_ROWS = 500032
NBT = BATCH // 128
IDX_PER_W = BATCH * HIST // NUM_WORKERS


def _transpose_body(t1_ref, t2_ref, o_ref):
    o_ref[...] = jnp.concatenate([t1_ref[...], t2_ref[...]], axis=0).T


_transpose = pl.pallas_call(
    _transpose_body,
    grid=(pl.cdiv(H_ROWS, TBLK),),
    in_specs=[
        pl.BlockSpec((D_MODEL, TBLK), lambda i: (0, i)),
        pl.BlockSpec((D_MODEL, TBLK), lambda i: (0, i + SPLIT // TBLK)),
    ],
    out_specs=pl.BlockSpec((TBLK, 128), lambda i: (i, 0)),
    out_shape=jax.ShapeDtypeStruct((H_ROWS, 128), jnp.float32),
)


def _format_body(in_ref, o_ref):
    for t2 in range(32):
        at2 = in_ref[0, t2 * 128:(t2 + 1) * 128, :].T
        for k in range(2):
            ck = at2[:, 64 * k:64 * k + 64]
            c = jnp.concatenate([ck[0:64], ck[64:128]], axis=1)
            for R in range(8):
                o_ref[0, R, t2 * 2 + k, :, :] = c[R * 8:(R + 1) * 8, :]


_format = pl.pallas_call(
    _format_body,
    grid=(HIST, NBT // 64),
    in_specs=[pl.BlockSpec((1, 4096, 128), lambda h, j: (h, j, 0))],
    out_specs=pl.BlockSpec((1, 8, 64, 8, 128), lambda h, j: (h, 0, j, 0, 0)),
    out_shape=jax.ShapeDtypeStruct((HIST, 8, NBT, 8, 128), jnp.float32),
)


def _make_gather():
    mesh = plsc.VectorSubcoreMesh(core_axis_name="c", subcore_axis_name="s")

    @functools.partial(
        pl.kernel,
        mesh=mesh,
        out_type=jax.ShapeDtypeStruct((BATCH * HIST, D_MODEL), jnp.float32),
        compiler_params=pltpu.CompilerParams(
            use_tc_tiling_on_sc=False, needs_layout_passes=False
        ),
        scratch_types=[
            pltpu.VMEM((IDX_PER_W,), jnp.int32),
            pltpu.VMEM((IDX_PER_W,), jnp.int32),
            pltpu.VMEM((NBUF, CHUNK, D_MODEL), jnp.float32),
            pltpu.SemaphoreType.DMA,
        ] + [pltpu.SemaphoreType.DMA] * (2 * NBUF),
    )
    def gather(table_hbm, idx_hbm, out_hbm, idx_v, idx2_v, rows_v,
               sem_i, *sems):
        sem_g = sems[:NBUF]
        sem_w = sems[NBUF:]
        wid = lax.axis_index("s") * 2 + lax.axis_index("c")
        base = wid * IDX_PER_W

        pltpu.async_copy(
            idx_hbm.at[pl.ds(base, IDX_PER_W)], idx_v, sem_i
        ).wait()

        iota = jnp.arange(16, dtype=jnp.int32)

        def remap_step(g, carry):
            j = iota + g * 16
            v = idx_v[pl.ds(g * 16, 16)]
            u = v + v
            u = jnp.where(v < SPLIT, u, u - (2 * SPLIT - 1))
            jj = lax.rem(j, 128)
            q = jj + jj - jnp.where(jj < 64, 0, 127)
            plsc.store_scatter(idx2_v, [j - jj + q], u)
            return carry

        lax.fori_loop(0, IDX_PER_W // 16, remap_step, 0)

        n_chunks = IDX_PER_W // CHUNK

        def g_copy(u, p):
            return pltpu.make_async_copy(
                table_hbm.at[idx2_v.at[pl.ds(u * CHUNK, CHUNK)]],
                rows_v.at[p], sem_g[p],
            )

        def w_copy(u, p):
            return pltpu.make_async_copy(
                rows_v.at[p], out_hbm.at[pl.ds(base + u * CHUNK, CHUNK)],
                sem_w[p],
            )

        def fire_gather(u, p):
            pltpu.async_copy(
                table_hbm.at[idx2_v.at[pl.ds(u * CHUNK, CHUNK)]],
                rows_v.at[p], sem_g[p],
            )

        def fire_write(u, p):
            pltpu.async_copy(
                rows_v.at[p], out_hbm.at[pl.ds(base + u * CHUNK, CHUNK)],
                sem_w[p],
            )

        fire_gather(0, 0)
        fire_gather(1, 1)

        def ring(k, carry):
            for j in range(NBUF):
                u = NBUF * k + j
                p = j
                pn = (j + 2) % NBUF

                @pl.when(u >= 2)
                def _():
                    w_copy(u - 2, pn).wait()

                @pl.when(u + 2 < n_chunks)
                def _():
                    fire_gather(u + 2, pn)

                g_copy(u, p).wait()
                fire_write(u, p)
            return carry

        lax.fori_loop(0, n_chunks // NBUF, ring, 0)
        w_copy(n_chunks - 2, (n_chunks - 2) % NBUF).wait()
        w_copy(n_chunks - 1, (n_chunks - 1) % NBUF).wait()

    return gather


@jax.jit
def kernel(x, embedding_matrix):
    idx = x.T.reshape(BATCH * HIST).astype(jnp.int32)
    t_view = embedding_matrix.T
    table_h = _transpose(t_view, t_view)
    table_rm = table_h.reshape(2 * H_ROWS, D_MODEL)
    rows = _make_gather()(table_rm, idx)
    out5 = _format(rows.reshape(HIST, BATCH // 2, 128))
    return out5.transpose(2, 4, 0, 1, 3).reshape(BATCH, HIST, D_MODEL)
